# baseline (speedup 1.0000x reference)
import functools

import jax
import jax.numpy as jnp
from jax import lax
from jax.experimental import pallas as pl
from jax.experimental.pallas import tpu as pltpu
from jax.experimental.pallas import tpu_sc as plsc

N = 10000
E = 320000
D = 128
DO = 40
DCA = 64
NPH = 2
D2 = 64

NC, NS = 2, 16
NW = NC * NS
NP = 10240
RPT = NP // NS

EC = 128
EROWS = 2560
CPW = EROWS // NW
EROWS_IDX = 2752
EPAD = EROWS_IDX * EC

_mesh = plsc.VectorSubcoreMesh(core_axis_name="c", subcore_axis_name="s")


K = 4
CF = 160
CS = 0
HC = 64


def _load_indices(c, s, h, src_hbm, dst_hbm, srcb, dstb):
    off = s * CF + c * (NS * CF - s * (CF - CS)) + h * HC
    pltpu.sync_copy(src_hbm.at[pl.ds(off, HC)], srcb)
    pltpu.sync_copy(dst_hbm.at[pl.ds(off, HC)], dstb)


def _run_halves(c, s, x_hbm, src_hbm, dst_hbm, srcb, dstb,
                rows, accS, sgA, sgB, ssA, ssB):
    gf = [8, 8, 4]
    gs = [0, 0, 0]
    for h in range(3):
        gi = gf[h] + c * (gs[h] - gf[h])
        _load_indices(c, s, h, src_hbm, dst_hbm, srcb, dstb)
        _edge_pipeline(gi, x_hbm, srcb, dstb, rows, accS, sgA, sgB, ssA, ssB)


def _edge_pipeline(gi, x_hbm, srcb, dstb, rows, accS, sgA, sgB, ssA, ssB):
    @pl.when(gi > 0)
    def _prologue():
        for k in range(K):
            pltpu.async_copy(x_hbm.at[srcb.at[k]], rows.at[k], sgA)
        for k in range(K):
            pltpu.async_copy(x_hbm.at[srcb.at[K + k]], rows.at[K + k], sgB)

    def body(i, carry):
        base = i * 2 * K
        for k in range(K):
            pltpu.make_async_copy(x_hbm.at[pl.ds(0, EC)], rows.at[k], sgA).wait()
        hA = [pltpu.async_copy(rows.at[k], accS.at[dstb.at[base + k]],
                               ssA, add=True) for k in range(K)]
        for k in range(K):
            pltpu.make_async_copy(x_hbm.at[pl.ds(0, EC)], rows.at[K + k], sgB).wait()
        hB = [pltpu.async_copy(rows.at[K + k], accS.at[dstb.at[base + K + k]],
                               ssB, add=True) for k in range(K)]
        for h in hA:
            h.wait()

        @pl.when(i < gi - 1)
        def _prefetch_a():
            for k in range(K):
                pltpu.async_copy(x_hbm.at[srcb.at[base + 2 * K + k]],
                                 rows.at[k], sgA)

        for h in hB:
            h.wait()

        @pl.when(i < gi - 1)
        def _prefetch_b():
            for k in range(K):
                pltpu.async_copy(x_hbm.at[srcb.at[base + 3 * K + k]],
                                 rows.at[K + k], sgB)

        return carry

    lax.fori_loop(0, gi, body, 0)


@functools.partial(
    pl.kernel,
    out_type=jax.ShapeDtypeStruct((NC, NPH, NP, DCA), jnp.float32),
    mesh=_mesh,
    scratch_types=[
        pltpu.VMEM((HC, EC), jnp.int32),
        pltpu.VMEM((HC, EC), jnp.int32),
        pltpu.VMEM((2 * K, EC, DCA), jnp.float32),
        pltpu.VMEM_SHARED((NP, DCA), jnp.float32),
        pltpu.SemaphoreType.DMA,
        pltpu.SemaphoreType.DMA,
        pltpu.SemaphoreType.DMA,
        pltpu.SemaphoreType.DMA,
    ],
    compiler_params=pltpu.CompilerParams(use_tc_tiling_on_sc=False),
)
def _sc_agg1(x0_hbm, x1_hbm, src_hbm, dst_hbm, zdc_hbm,
             agg_out, srcb, dstb, rows, accS, sgA, sgB, ssA, ssB):
    c = lax.axis_index("c")
    s = lax.axis_index("s")
    for p, x_hbm in enumerate((x0_hbm, x1_hbm)):
        pltpu.sync_copy(zdc_hbm, accS.at[pl.ds(s * RPT, RPT)])
        plsc.subcore_barrier()
        _run_halves(c, s, x_hbm, src_hbm, dst_hbm, srcb, dstb,
                    rows, accS, sgA, sgB, ssA, ssB)
        plsc.subcore_barrier()
        pltpu.sync_copy(accS.at[pl.ds(s * RPT, RPT)],
                        agg_out.at[c, p, pl.ds(s * RPT, RPT)])


@functools.partial(
    pl.kernel,
    out_type=jax.ShapeDtypeStruct((NC, NP, 16), jnp.float32),
    mesh=_mesh,
    scratch_types=[
        pltpu.VMEM((CPW, EC), jnp.int32),
        pltpu.VMEM((EC, 16), jnp.float32),
        pltpu.VMEM_SHARED((NP, 16), jnp.float32),
    ],
    compiler_params=pltpu.CompilerParams(use_tc_tiling_on_sc=False),
)
def _sc_deg(dst_hbm, z16_hbm, ones_hbm, deg_out, dstb, onesb, degS):
    c = lax.axis_index("c")
    s = lax.axis_index("s")
    w = s * NC + c
    pltpu.sync_copy(ones_hbm, onesb)
    pltpu.sync_copy(dst_hbm.at[pl.ds(w * CPW, CPW)], dstb)
    pltpu.sync_copy(z16_hbm, degS.at[pl.ds(s * RPT, RPT)])
    plsc.subcore_barrier()

    def body(j, carry):
        pltpu.sync_copy(onesb, degS.at[dstb.at[j]], add=True)
        return carry

    lax.fori_loop(0, CPW, body, 0)
    plsc.subcore_barrier()
    pltpu.sync_copy(degS.at[pl.ds(s * RPT, RPT)],
                    deg_out.at[c, pl.ds(s * RPT, RPT)])


@functools.partial(
    pl.kernel,
    out_type=jax.ShapeDtypeStruct((NC, NP, D2), jnp.float32),
    mesh=_mesh,
    scratch_types=[
        pltpu.VMEM((HC, EC), jnp.int32),
        pltpu.VMEM((HC, EC), jnp.int32),
        pltpu.VMEM((2 * K, EC, D2), jnp.float32),
        pltpu.VMEM_SHARED((NP, D2), jnp.float32),
        pltpu.SemaphoreType.DMA,
        pltpu.SemaphoreType.DMA,
        pltpu.SemaphoreType.DMA,
        pltpu.SemaphoreType.DMA,
    ],
    compiler_params=pltpu.CompilerParams(use_tc_tiling_on_sc=False),
)
def _sc_agg2(g_hbm, src_hbm, dst_hbm, z2_hbm,
             agg_out, srcb, dstb, rows, accS, sgA, sgB, ssA, ssB):
    c = lax.axis_index("c")
    s = lax.axis_index("s")
    pltpu.sync_copy(z2_hbm, accS.at[pl.ds(s * RPT, RPT)])
    plsc.subcore_barrier()
    _run_halves(c, s, g_hbm, src_hbm, dst_hbm, srcb, dstb,
                rows, accS, sgA, sgB, ssA, ssB)
    plsc.subcore_barrier()
    pltpu.sync_copy(accS.at[pl.ds(s * RPT, RPT)],
                    agg_out.at[c, pl.ds(s * RPT, RPT)])


_R = 1024


def _tc1_body(hs_r, a_r, d_r, wn1_r, wn2_r, ws2_r, b2_r, g_r, os_r):
    a = a_r[...]
    d = d_r[...]
    deg = jnp.maximum(d[0, :, :1] + d[1, :, :1], 1.0)
    hn = jnp.concatenate(
        [a[0, 0] + a[1, 0], a[0, 1] + a[1, 1]], axis=1) / deg
    h = (hs_r[...]
         + jnp.dot(hn, wn1_r[...], preferred_element_type=jnp.float32))
    h = jnp.maximum(h, 0.0)
    g_r[...] = jnp.dot(h, wn2_r[...], preferred_element_type=jnp.float32)
    os_r[...] = (jnp.dot(h, ws2_r[...], preferred_element_type=jnp.float32)
                 + b2_r[...])


def _tc_self_body(x_r, ws1_r, b1_r, hs_r):
    hs_r[...] = (jnp.dot(x_r[...], ws1_r[...], preferred_element_type=jnp.float32)
                 + b1_r[...])


def _tc2_body(os_r, a_r, d_r, o_r):
    a = a_r[...]
    d = d_r[...]
    deg = jnp.maximum(d[0, :, :1] + d[1, :, :1], 1.0)
    o_r[...] = os_r[...] + (a[0] + a[1]) / deg


def _row_spec(d):
    return pl.BlockSpec((_R, d), lambda i: (i, 0))


def _full_spec(r, c):
    return pl.BlockSpec((r, c), lambda i: (0, 0))


def kernel(feat, edge_index, W_self1, W_neigh1, b1, W_self2, W_neigh2, b2):
    src = edge_index[0].astype(jnp.int32)
    dst = edge_index[1].astype(jnp.int32)
    src2d = jnp.concatenate(
        [src, jnp.zeros((EPAD - E,), jnp.int32)]).reshape(EROWS_IDX, EC)
    dst2d = jnp.concatenate(
        [dst, jnp.full((EPAD - E,), N, jnp.int32)]).reshape(EROWS_IDX, EC)
    x_pad = jnp.pad(feat, ((0, NP - N), (0, 0)))
    xs = [x_pad[:, p * DCA:(p + 1) * DCA] for p in range(NPH)]

    zdc = jnp.zeros((RPT, DCA), jnp.float32)
    z2 = jnp.zeros((RPT, D2), jnp.float32)
    z16 = jnp.zeros((RPT, 16), jnp.float32)
    ones16 = jnp.zeros((EC, 16), jnp.float32).at[:, 0].set(1.0)

    agg1 = _sc_agg1(xs[0], xs[1], src2d, dst2d, zdc)
    deg = _sc_deg(dst2d, z16, ones16)

    wn2p = jnp.zeros((D, D2), jnp.float32).at[:, :DO].set(W_neigh2)
    ws2p = jnp.zeros((D, D2), jnp.float32).at[:, :DO].set(W_self2)
    b2p = jnp.zeros((1, D2), jnp.float32).at[0, :DO].set(b2)
    b1r = b1.reshape(1, D)

    grid = (NP // _R,)
    h_self = pl.pallas_call(
        _tc_self_body,
        grid=grid,
        in_specs=[_row_spec(D), _full_spec(D, D), _full_spec(1, D)],
        out_specs=_row_spec(D),
        out_shape=jax.ShapeDtypeStruct((NP, D), jnp.float32),
    )(x_pad, W_self1, b1r)

    g, o_self = pl.pallas_call(
        _tc1_body,
        grid=grid,
        in_specs=[
            _row_spec(D),
            pl.BlockSpec((NC, NPH, _R, DCA), lambda i: (0, 0, i, 0)),
            pl.BlockSpec((NC, _R, 16), lambda i: (0, i, 0)),
            _full_spec(D, D), _full_spec(D, D2),
            _full_spec(D, D2), _full_spec(1, D2),
        ],
        out_specs=[_row_spec(D2), _row_spec(D2)],
        out_shape=[
            jax.ShapeDtypeStruct((NP, D2), jnp.float32),
            jax.ShapeDtypeStruct((NP, D2), jnp.float32),
        ],
    )(h_self, agg1, deg, W_neigh1, wn2p, ws2p, b2p)

    agg2 = _sc_agg2(g, src2d, dst2d, z2)

    out = pl.pallas_call(
        _tc2_body,
        grid=grid,
        in_specs=[
            _row_spec(D2),
            pl.BlockSpec((NC, _R, D2), lambda i: (0, i, 0)),
            pl.BlockSpec((NC, _R, 16), lambda i: (0, i, 0)),
        ],
        out_specs=_row_spec(D2),
        out_shape=jax.ShapeDtypeStruct((NP, D2), jnp.float32),
    )(o_self, agg2, deg)

    return out[:N, :DO]


# speedup vs baseline: 1.3826x; 1.3826x over previous
import functools

import jax
import jax.numpy as jnp
from jax import lax
from jax.experimental import pallas as pl
from jax.experimental.pallas import tpu as pltpu
from jax.experimental.pallas import tpu_sc as plsc

N = 10000
E = 320000
D = 128
DO = 40
DCA = 64
NPH = 2
D2 = 64

NC, NS = 2, 16
NW = NC * NS
NP = 10240
RPT = NP // NS

EC = 128
EROWS = 2560
CPW = EROWS // NW
EROWS_IDX = 2752
EPAD = EROWS_IDX * EC

_mesh = plsc.VectorSubcoreMesh(core_axis_name="c", subcore_axis_name="s")


K = 4
CF = 152
CS = 8
HC = 64


def _load_indices(c, s, h, src_hbm, dst_hbm, srcb, dstb):
    off = s * CF + c * (NS * CF - s * (CF - CS)) + h * HC
    pltpu.sync_copy(src_hbm.at[pl.ds(off, HC)], srcb)
    pltpu.sync_copy(dst_hbm.at[pl.ds(off, HC)], dstb)


def _run_halves(c, s, x_hbm, src_hbm, dst_hbm, srcb, dstb,
                rows, accS, sgA, sgB, ssA, ssB):
    gf = [8, 8, 3]
    gs = [1, 0, 0]
    for h in range(3):
        gi = gf[h] + c * (gs[h] - gf[h])
        _load_indices(c, s, h, src_hbm, dst_hbm, srcb, dstb)
        _edge_pipeline(gi, x_hbm, srcb, dstb, rows, accS, sgA, sgB, ssA, ssB)


def _edge_pipeline(gi, x_hbm, srcb, dstb, rows, accS, sgA, sgB, ssA, ssB):
    @pl.when(gi > 0)
    def _prologue():
        for k in range(K):
            pltpu.async_copy(x_hbm.at[srcb.at[k]], rows.at[k], sgA)
        for k in range(K):
            pltpu.async_copy(x_hbm.at[srcb.at[K + k]], rows.at[K + k], sgB)

    def body(i, carry):
        base = i * 2 * K
        for k in range(K):
            pltpu.make_async_copy(x_hbm.at[pl.ds(0, EC)], rows.at[k], sgA).wait()
        hA = [pltpu.async_copy(rows.at[k], accS.at[dstb.at[base + k]],
                               ssA, add=True) for k in range(K)]
        for k in range(K):
            pltpu.make_async_copy(x_hbm.at[pl.ds(0, EC)], rows.at[K + k], sgB).wait()
        hB = [pltpu.async_copy(rows.at[K + k], accS.at[dstb.at[base + K + k]],
                               ssB, add=True) for k in range(K)]
        for h in hA:
            h.wait()

        @pl.when(i < gi - 1)
        def _prefetch_a():
            for k in range(K):
                pltpu.async_copy(x_hbm.at[srcb.at[base + 2 * K + k]],
                                 rows.at[k], sgA)

        for h in hB:
            h.wait()

        @pl.when(i < gi - 1)
        def _prefetch_b():
            for k in range(K):
                pltpu.async_copy(x_hbm.at[srcb.at[base + 3 * K + k]],
                                 rows.at[K + k], sgB)

        return carry

    lax.fori_loop(0, gi, body, 0)


@functools.partial(
    pl.kernel,
    out_type=jax.ShapeDtypeStruct((NC, NPH, NP, DCA), jnp.float32),
    mesh=_mesh,
    scratch_types=[
        pltpu.VMEM((HC, EC), jnp.int32),
        pltpu.VMEM((HC, EC), jnp.int32),
        pltpu.VMEM((2 * K, EC, DCA), jnp.float32),
        pltpu.VMEM_SHARED((NP, DCA), jnp.float32),
        pltpu.SemaphoreType.DMA,
        pltpu.SemaphoreType.DMA,
        pltpu.SemaphoreType.DMA,
        pltpu.SemaphoreType.DMA,
    ],
    compiler_params=pltpu.CompilerParams(use_tc_tiling_on_sc=False),
)
def _sc_agg1(x0_hbm, x1_hbm, src_hbm, dst_hbm, zdc_hbm,
             agg_out, srcb, dstb, rows, accS, sgA, sgB, ssA, ssB):
    c = lax.axis_index("c")
    s = lax.axis_index("s")
    for p, x_hbm in enumerate((x0_hbm, x1_hbm)):
        pltpu.sync_copy(zdc_hbm, accS.at[pl.ds(s * RPT, RPT)])
        plsc.subcore_barrier()
        _run_halves(c, s, x_hbm, src_hbm, dst_hbm, srcb, dstb,
                    rows, accS, sgA, sgB, ssA, ssB)
        plsc.subcore_barrier()
        pltpu.sync_copy(accS.at[pl.ds(s * RPT, RPT)],
                        agg_out.at[c, p, pl.ds(s * RPT, RPT)])


@functools.partial(
    pl.kernel,
    out_type=jax.ShapeDtypeStruct((NC, NP, 16), jnp.float32),
    mesh=_mesh,
    scratch_types=[
        pltpu.VMEM((CPW, EC), jnp.int32),
        pltpu.VMEM((EC, 16), jnp.float32),
        pltpu.VMEM_SHARED((NP, 16), jnp.float32),
    ],
    compiler_params=pltpu.CompilerParams(use_tc_tiling_on_sc=False),
)
def _sc_deg(dst_hbm, z16_hbm, ones_hbm, deg_out, dstb, onesb, degS):
    c = lax.axis_index("c")
    s = lax.axis_index("s")
    w = s * NC + c
    pltpu.sync_copy(ones_hbm, onesb)
    pltpu.sync_copy(dst_hbm.at[pl.ds(w * CPW, CPW)], dstb)
    pltpu.sync_copy(z16_hbm, degS.at[pl.ds(s * RPT, RPT)])
    plsc.subcore_barrier()

    def body(j, carry):
        pltpu.sync_copy(onesb, degS.at[dstb.at[j]], add=True)
        return carry

    lax.fori_loop(0, CPW, body, 0)
    plsc.subcore_barrier()
    pltpu.sync_copy(degS.at[pl.ds(s * RPT, RPT)],
                    deg_out.at[c, pl.ds(s * RPT, RPT)])


@functools.partial(
    pl.kernel,
    out_type=jax.ShapeDtypeStruct((NC, NP, D2), jnp.float32),
    mesh=_mesh,
    scratch_types=[
        pltpu.VMEM((HC, EC), jnp.int32),
        pltpu.VMEM((HC, EC), jnp.int32),
        pltpu.VMEM((2 * K, EC, D2), jnp.float32),
        pltpu.VMEM_SHARED((NP, D2), jnp.float32),
        pltpu.SemaphoreType.DMA,
        pltpu.SemaphoreType.DMA,
        pltpu.SemaphoreType.DMA,
        pltpu.SemaphoreType.DMA,
    ],
    compiler_params=pltpu.CompilerParams(use_tc_tiling_on_sc=False),
)
def _sc_agg2(g_hbm, src_hbm, dst_hbm, z2_hbm,
             agg_out, srcb, dstb, rows, accS, sgA, sgB, ssA, ssB):
    c = lax.axis_index("c")
    s = lax.axis_index("s")
    pltpu.sync_copy(z2_hbm, accS.at[pl.ds(s * RPT, RPT)])
    plsc.subcore_barrier()
    _run_halves(c, s, g_hbm, src_hbm, dst_hbm, srcb, dstb,
                rows, accS, sgA, sgB, ssA, ssB)
    plsc.subcore_barrier()
    pltpu.sync_copy(accS.at[pl.ds(s * RPT, RPT)],
                    agg_out.at[c, pl.ds(s * RPT, RPT)])


_R = 1024


def _tc1_body(hs_r, a_r, d_r, wn1_r, wn2_r, ws2_r, b2_r, g_r, os_r):
    a = a_r[...]
    d = d_r[...]
    deg = jnp.maximum(d[0, :, :1] + d[1, :, :1], 1.0)
    hn = jnp.concatenate(
        [a[0, 0] + a[1, 0], a[0, 1] + a[1, 1]], axis=1) / deg
    h = (hs_r[...]
         + jnp.dot(hn, wn1_r[...], preferred_element_type=jnp.float32))
    h = jnp.maximum(h, 0.0)
    g_r[...] = jnp.dot(h, wn2_r[...], preferred_element_type=jnp.float32)
    os_r[...] = (jnp.dot(h, ws2_r[...], preferred_element_type=jnp.float32)
                 + b2_r[...])


def _tc_self_body(x_r, ws1_r, b1_r, hs_r):
    hs_r[...] = (jnp.dot(x_r[...], ws1_r[...], preferred_element_type=jnp.float32)
                 + b1_r[...])


def _tc2_body(os_r, a_r, d_r, o_r):
    a = a_r[...]
    d = d_r[...]
    deg = jnp.maximum(d[0, :, :1] + d[1, :, :1], 1.0)
    o_r[...] = os_r[...] + (a[0] + a[1]) / deg


def _row_spec(d):
    return pl.BlockSpec((_R, d), lambda i: (i, 0))


def _full_spec(r, c):
    return pl.BlockSpec((r, c), lambda i: (0, 0))


def kernel(feat, edge_index, W_self1, W_neigh1, b1, W_self2, W_neigh2, b2):
    src = edge_index[0].astype(jnp.int32)
    dst = edge_index[1].astype(jnp.int32)
    src2d = jnp.concatenate(
        [src, jnp.zeros((EPAD - E,), jnp.int32)]).reshape(EROWS_IDX, EC)
    dst2d = jnp.concatenate(
        [dst, jnp.full((EPAD - E,), N, jnp.int32)]).reshape(EROWS_IDX, EC)
    x_pad = jnp.pad(feat, ((0, NP - N), (0, 0)))
    xs = [x_pad[:, p * DCA:(p + 1) * DCA] for p in range(NPH)]

    zdc = jnp.zeros((RPT, DCA), jnp.float32)
    z2 = jnp.zeros((RPT, D2), jnp.float32)
    z16 = jnp.zeros((RPT, 16), jnp.float32)
    ones16 = jnp.zeros((EC, 16), jnp.float32).at[:, 0].set(1.0)

    agg1 = _sc_agg1(xs[0], xs[1], src2d, dst2d, zdc)
    deg = _sc_deg(dst2d, z16, ones16)

    wn2p = jnp.zeros((D, D2), jnp.float32).at[:, :DO].set(W_neigh2)
    ws2p = jnp.zeros((D, D2), jnp.float32).at[:, :DO].set(W_self2)
    b2p = jnp.zeros((1, D2), jnp.float32).at[0, :DO].set(b2)
    b1r = b1.reshape(1, D)

    grid = (NP // _R,)
    h_self = pl.pallas_call(
        _tc_self_body,
        grid=grid,
        in_specs=[_row_spec(D), _full_spec(D, D), _full_spec(1, D)],
        out_specs=_row_spec(D),
        out_shape=jax.ShapeDtypeStruct((NP, D), jnp.float32),
    )(x_pad, W_self1, b1r)

    g, o_self = pl.pallas_call(
        _tc1_body,
        grid=grid,
        in_specs=[
            _row_spec(D),
            pl.BlockSpec((NC, NPH, _R, DCA), lambda i: (0, 0, i, 0)),
            pl.BlockSpec((NC, _R, 16), lambda i: (0, i, 0)),
            _full_spec(D, D), _full_spec(D, D2),
            _full_spec(D, D2), _full_spec(1, D2),
        ],
        out_specs=[_row_spec(D2), _row_spec(D2)],
        out_shape=[
            jax.ShapeDtypeStruct((NP, D2), jnp.float32),
            jax.ShapeDtypeStruct((NP, D2), jnp.float32),
        ],
    )(h_self, agg1, deg, W_neigh1, wn2p, ws2p, b2p)

    agg2 = _sc_agg2(g, src2d, dst2d, z2)

    out = pl.pallas_call(
        _tc2_body,
        grid=grid,
        in_specs=[
            _row_spec(D2),
            pl.BlockSpec((NC, _R, D2), lambda i: (0, i, 0)),
            pl.BlockSpec((NC, _R, 16), lambda i: (0, i, 0)),
        ],
        out_specs=_row_spec(D2),
        out_shape=jax.ShapeDtypeStruct((NP, D2), jnp.float32),
    )(o_self, agg2, deg)

    return out[:N, :DO]
